# Initial kernel scaffold; baseline (speedup 1.0000x reference)
import jax
import jax.numpy as jnp
from jax.experimental import pallas as pl


def kernel(input_ids, tok_table, pos_table):
    raise NotImplementedError("write your pallas kernel here")


# speedup vs baseline: 1.2330x; 1.2330x over previous
import functools

import jax
import jax.numpy as jnp
from jax import lax
from jax.experimental import pallas as pl
from jax.experimental.pallas import tpu as pltpu
from jax.experimental.pallas import tpu_sc as plsc


def _build_sc_kernel(B, S, V, H, C):
    info = plsc.get_sparse_core_info()
    NC, NS, L = info.num_cores, info.num_subcores, info.num_lanes
    NW = NC * NS
    SW = S // NW
    assert S % NW == 0 and SW % C == 0 and H % L == 0

    mesh = plsc.VectorSubcoreMesh(core_axis_name="c", subcore_axis_name="s")

    @functools.partial(
        pl.kernel,
        mesh=mesh,
        out_type=jax.ShapeDtypeStruct((B * S, H), jnp.float32),
        scratch_types=[
            pltpu.VMEM((C,), jnp.int32),
            pltpu.VMEM((C, H), jnp.float32),
            pltpu.VMEM((C, H), jnp.float32),
            pltpu.SemaphoreType.DMA,
        ],
    )
    def k(ids_hbm, tok_hbm, pos_hbm, out_hbm, idx_v, pos_v, tok_v, sem):
        wid = lax.axis_index("s") * NC + lax.axis_index("c")
        s_base = wid * SW
        for j in range(SW // C):
            s0 = s_base + j * C
            pltpu.sync_copy(pos_hbm.at[pl.ds(s0, C)], pos_v)
            for b in range(B):
                flat0 = b * S + s0
                pltpu.sync_copy(ids_hbm.at[pl.ds(flat0, C)], idx_v)
                pltpu.async_copy(tok_hbm.at[idx_v], tok_v, sem).wait()

                def body(r, carry):
                    for kk in range(H // L):
                        sl = pl.ds(kk * L, L)
                        tok_v[r, sl] = tok_v[r, sl] + pos_v[r, sl]
                    return carry

                lax.fori_loop(0, C, body, 0)
                pltpu.sync_copy(tok_v, out_hbm.at[pl.ds(flat0, C)])

    return k


def kernel(input_ids, tok_table, pos_table):
    B, S = input_ids.shape
    V, H = tok_table.shape
    ids_flat = input_ids.reshape(B * S).astype(jnp.int32)
    k = _build_sc_kernel(B, S, V, H, C=32)
    out = k(ids_flat, tok_table, pos_table)
    return out.reshape(B, S, H)


# speedup vs baseline: 1.2440x; 1.0090x over previous
import functools

import jax
import jax.numpy as jnp
from jax import lax
from jax.experimental import pallas as pl
from jax.experimental.pallas import tpu as pltpu
from jax.experimental.pallas import tpu_sc as plsc


def _build_sc_kernel(B, S, V, H, C, NBUF):
    info = plsc.get_sparse_core_info()
    NC, NS, L = info.num_cores, info.num_subcores, info.num_lanes
    NW = NC * NS
    SW = S // NW
    assert S % NW == 0 and SW % C == 0 and H % L == 0
    steps_per_b = SW // C
    nsteps = B * steps_per_b

    mesh = plsc.VectorSubcoreMesh(core_axis_name="c", subcore_axis_name="s")

    @functools.partial(
        pl.kernel,
        mesh=mesh,
        out_type=jax.ShapeDtypeStruct((B * S, H), jnp.float32),
        scratch_types=[
            pltpu.VMEM((B * SW,), jnp.int32),
            pltpu.VMEM((SW, H), jnp.float32),
            pltpu.VMEM((NBUF, C, H), jnp.float32),
            pltpu.SemaphoreType.DMA,
            pltpu.SemaphoreType.DMA,
        ],
    )
    def k(ids_hbm, tok_hbm, pos_hbm, out_hbm, idx_v, pos_v, tok_v, gsem, ssem):
        wid = lax.axis_index("s") * NC + lax.axis_index("c")
        s_base = wid * SW
        for b in range(B):
            pltpu.sync_copy(ids_hbm.at[pl.ds(b * S + s_base, SW)],
                            idx_v.at[pl.ds(b * SW, SW)])
        pltpu.sync_copy(pos_hbm.at[pl.ds(s_base, SW)], pos_v)

        def step_slices(t):
            b, j = divmod(t, steps_per_b)
            idx_sl = pl.ds(b * SW + j * C, C)
            out_sl = pl.ds(b * S + s_base + j * C, C)
            return idx_sl, out_sl, j * C

        def start_gather(t):
            idx_sl, _, _ = step_slices(t)
            return pltpu.async_copy(tok_hbm.at[idx_v.at[idx_sl]],
                                    tok_v.at[t % NBUF], gsem)

        def start_store(t):
            _, out_sl, _ = step_slices(t)
            return pltpu.async_copy(tok_v.at[t % NBUF], out_hbm.at[out_sl], ssem)

        LOOK = NBUF - 1
        gathers = {}
        stores = {}
        for t in range(min(LOOK, nsteps)):
            gathers[t] = start_gather(t)
        for t in range(nsteps):
            nxt = t + LOOK
            if nxt < nsteps:
                if nxt >= NBUF:
                    stores[nxt - NBUF].wait()
                gathers[nxt] = start_gather(nxt)
            gathers[t].wait()
            _, _, p0 = step_slices(t)

            def body(r, carry):
                buf = t % NBUF
                for kk in range(H // L):
                    sl = pl.ds(kk * L, L)
                    tok_v[buf, r, sl] = tok_v[buf, r, sl] + pos_v[p0 + r, sl]
                return carry

            lax.fori_loop(0, C, body, 0)
            stores[t] = start_store(t)
        for t in range(max(0, nsteps - NBUF), nsteps):
            if t in stores:
                stores[t].wait()

    return k


def kernel(input_ids, tok_table, pos_table):
    B, S = input_ids.shape
    V, H = tok_table.shape
    ids_flat = input_ids.reshape(B * S).astype(jnp.int32)
    k = _build_sc_kernel(B, S, V, H, C=16, NBUF=3)
    out = k(ids_flat, tok_table, pos_table)
    return out.reshape(B, S, H)


# speedup vs baseline: 1.9404x; 1.5598x over previous
import jax
import jax.numpy as jnp
from jax import lax
from jax.experimental import pallas as pl
from jax.experimental.pallas import tpu as pltpu
from jax.experimental.pallas import tpu_sc as plsc


def _build_sc_kernel(B, S, V, H, C, NBUF, LOOK, HALF):
    info = plsc.get_sparse_core_info()
    NC, NS, L = info.num_cores, info.num_subcores, info.num_lanes
    NW = NC * NS
    SW = S // NW
    assert S % NW == 0 and SW % HALF == 0 and HALF % C == 0 and H % L == 0
    cph = HALF // C
    steps_per_half = B * cph
    nsteps = (SW // HALF) * steps_per_half

    mesh = plsc.VectorSubcoreMesh(core_axis_name="c", subcore_axis_name="s")

    import functools

    @functools.partial(
        pl.kernel,
        mesh=mesh,
        out_type=jax.ShapeDtypeStruct((B * S, H), jnp.float32),
        scratch_types=[
            pltpu.VMEM((B * SW,), jnp.int32),
            pltpu.VMEM((HALF, H), jnp.float32),
            pltpu.VMEM((NBUF, C, H), jnp.float32),
            pltpu.SemaphoreType.DMA,
            pltpu.SemaphoreType.DMA,
        ],
    )
    def k(ids_hbm, tok_hbm, pos_hbm, out_hbm, idx_v, pos_v, tok_v, gsem, ssem):
        wid = lax.axis_index("s") * NC + lax.axis_index("c")
        s_base = wid * SW
        for b in range(B):
            pltpu.sync_copy(ids_hbm.at[pl.ds(b * S + s_base, SW)],
                            idx_v.at[pl.ds(b * SW, SW)])

        def coords(t):
            j, r = divmod(t, steps_per_half)
            b, jj = divmod(r, cph)
            return j, b, jj

        def gather_desc(t):
            j, b, jj = coords(t)
            off = j * HALF + jj * C
            return pltpu.make_async_copy(
                tok_hbm.at[idx_v.at[pl.ds(b * SW + off, C)]],
                tok_v.at[t % NBUF], gsem)

        def store_desc(t):
            j, b, jj = coords(t)
            return pltpu.make_async_copy(
                tok_v.at[t % NBUF],
                out_hbm.at[pl.ds(b * S + s_base + j * HALF + jj * C, C)],
                ssem)

        for t in range(LOOK):
            gather_desc(t).start()

        def body(t, carry):
            nxt = t + LOOK

            @pl.when(jnp.logical_and(nxt >= NBUF, nxt < nsteps))
            def _():
                store_desc(nxt - NBUF).wait()

            @pl.when(nxt < nsteps)
            def _():
                gather_desc(nxt).start()

            @pl.when(t % steps_per_half == 0)
            def _():
                j = t // steps_per_half
                pltpu.sync_copy(pos_hbm.at[pl.ds(s_base + j * HALF, HALF)],
                                pos_v)

            gather_desc(t).wait()
            buf = t % NBUF
            _, _, jj = coords(t)
            p0 = jj * C

            @plsc.parallel_loop(0, C)
            def _add(r):
                for kk in range(H // L):
                    sl = pl.ds(kk * L, L)
                    tok_v[buf, r, sl] = tok_v[buf, r, sl] + pos_v[p0 + r, sl]

            store_desc(t).start()
            return carry

        lax.fori_loop(0, nsteps, body, 0)
        for t in range(nsteps - NBUF, nsteps):
            store_desc(t).wait()

    return k


def kernel(input_ids, tok_table, pos_table):
    B, S = input_ids.shape
    V, H = tok_table.shape
    ids_flat = input_ids.reshape(B * S).astype(jnp.int32)
    k = _build_sc_kernel(B, S, V, H, C=16, NBUF=4, LOOK=2, HALF=32)
    out = k(ids_flat, tok_table, pos_table)
    return out.reshape(B, S, H)
